# Initial kernel scaffold; baseline (speedup 1.0000x reference)
import functools

import jax
import jax.numpy as jnp
from jax import lax
from jax.experimental import pallas as pl
from jax.experimental.pallas import tpu as pltpu
from jax.experimental.pallas import tpu_sc as plsc

_N = 10000
_D = 128
_E = 320000
_CHUNK = 128
_NSC = 2
_NTILE = 16
_NW = _NSC * _NTILE
_NP = 10240
_NCHUNK = -(-_E // _CHUNK)
_NCHUNK = -(-_NCHUNK // _NTILE) * _NTILE
_EP = _NCHUNK * _CHUNK
_TCH = _NCHUNK // _NTILE
_HR = _NP // 16
_NODES_PER_W = _NP // _NW
_ROWS_PER_W = _NODES_PER_W // 16


def _sc_mesh():
    return plsc.VectorSubcoreMesh(core_axis_name="c", subcore_axis_name="s")


def _deg_body(col_hbm, dis_hbm, col_v, hist, rowidx, dloc, ebuf, acc, sem):
    c = lax.axis_index("c")
    s = lax.axis_index("s")
    w = c * _NTILE + s

    pltpu.sync_copy(col_hbm.at[pl.ds(s * _TCH, _TCH)], col_v)

    zeros16 = jnp.zeros((16,), jnp.float32)

    def _zh(i, carry):
        hist[i, :] = zeros16
        return carry

    lax.fori_loop(0, _HR, _zh, 0)
    nrow = _HR // _NTILE
    pltpu.sync_copy(hist.at[pl.ds(s * nrow, nrow)], acc.at[pl.ds(s * nrow, nrow)])

    ones16 = jnp.ones((16,), jnp.float32)

    def _hb(r, carry):
        for m in range(8):
            idx = col_v[r, pl.ds(m * 16, 16)]
            plsc.addupdate_scatter(
                hist, [lax.shift_right_logical(idx, 4), idx & 15], ones16)
        return carry

    lax.fori_loop(0, _TCH, _hb, 0)

    iota16 = lax.broadcasted_iota(jnp.int32, (16,), 0)
    nblk = _HR // _CHUNK
    for j in range(nblk):
        for m in range(8):
            rowidx[j, pl.ds(m * 16, 16)] = iota16 + (j * _CHUNK + m * 16)

    plsc.subcore_barrier()

    for j in range(nblk):
        pltpu.sync_copy(hist.at[pl.ds(j * _CHUNK, _CHUNK)],
                        acc.at[rowidx.at[j]], add=True)

    plsc.subcore_barrier()

    pltpu.sync_copy(acc.at[pl.ds(w * _ROWS_PER_W, _ROWS_PER_W)], dloc)

    def _rs(i, carry):
        d = dloc[i, :] + 1.0
        bits = lax.bitcast_convert_type(d, jnp.int32)
        y = lax.bitcast_convert_type(
            0x5F3759DF - lax.shift_right_logical(bits, 1), jnp.float32)
        y = y * (1.5 - 0.5 * d * y * y)
        y = y * (1.5 - 0.5 * d * y * y)
        y = y * (1.5 - 0.5 * d * y * y)
        dloc[i, :] = y
        return carry

    lax.fori_loop(0, _ROWS_PER_W, _rs, 0)

    zi16 = jnp.zeros((16,), jnp.int32)

    def _ex(i, carry):
        for l in range(16):
            v = plsc.load_gather(dloc, [zi16 + i, zi16 + l])
            for m in range(8):
                ebuf[i * 16 + l, pl.ds(m * 16, 16)] = v
        return carry

    lax.fori_loop(0, _ROWS_PER_W, _ex, 0)
    pltpu.sync_copy(ebuf, dis_hbm.at[pl.ds(w * _NODES_PER_W, _NODES_PER_W)])


def _deg(col2d):
    fn = functools.partial(
        pl.kernel,
        out_type=jax.ShapeDtypeStruct((_NP, _D), jnp.float32),
        mesh=_sc_mesh(),
        scratch_types=[
            pltpu.VMEM((_TCH, _CHUNK), jnp.int32),
            pltpu.VMEM((_HR, 16), jnp.float32),
            pltpu.VMEM((_HR // _CHUNK, _CHUNK), jnp.int32),
            pltpu.VMEM((_ROWS_PER_W, 16), jnp.float32),
            pltpu.VMEM((_NODES_PER_W, _D), jnp.float32),
            pltpu.VMEM_SHARED((_HR, 16), jnp.float32),
            pltpu.SemaphoreType.DMA,
        ],
    )(_deg_body)
    return fn(col2d)


def _mm_body(x_ref, d_ref, wm_ref, wl_ref, omu_ref, ols_ref):
    xs = x_ref[...] * d_ref[...]
    omu_ref[...] = jnp.dot(xs, wm_ref[...], preferred_element_type=jnp.float32)
    ols_ref[...] = jnp.dot(xs, wl_ref[...], preferred_element_type=jnp.float32)


def _mm(x_pad, dis_wide, W_mu, W_ls):
    br = 1024
    return pl.pallas_call(
        _mm_body,
        grid=(_NP // br,),
        in_specs=[
            pl.BlockSpec((br, _D), lambda i: (i, 0)),
            pl.BlockSpec((br, _D), lambda i: (i, 0)),
            pl.BlockSpec((_D, _D), lambda i: (0, 0)),
            pl.BlockSpec((_D, _D), lambda i: (0, 0)),
        ],
        out_specs=[pl.BlockSpec((br, _D), lambda i: (i, 0))] * 2,
        out_shape=[jax.ShapeDtypeStruct((_NP, _D), jnp.float32)] * 2,
    )(x_pad, dis_wide, W_mu, W_ls)


def _agg_body(xmu_hbm, xls_hbm, row_hbm, col_hbm, omu_hbm, ols_hbm,
              row_v, col_v, buf, acc, sem):
    c = lax.axis_index("c")
    s = lax.axis_index("s")

    pltpu.sync_copy(row_hbm.at[pl.ds(s * _TCH, _TCH)], row_v)
    pltpu.sync_copy(col_hbm.at[pl.ds(s * _TCH, _TCH)], col_v)

    zeros16 = jnp.zeros((16,), jnp.float32)

    def _zb(i, carry):
        for m in range(8):
            buf[i, pl.ds(m * 16, 16)] = zeros16
        return carry

    lax.fori_loop(0, _CHUNK, _zb, 0)
    nblk = (_NP // _NTILE) // _CHUNK
    for j in range(nblk):
        pltpu.sync_copy(buf, acc.at[pl.ds((s * nblk + j) * _CHUNK, _CHUNK)])

    plsc.subcore_barrier()

    def _run(x_hbm):
        def _body(k, carry):
            pltpu.async_copy(x_hbm.at[row_v.at[k]], buf, sem).wait()
            pltpu.sync_copy(buf, acc.at[col_v.at[k]], add=True)
            return carry
        lax.fori_loop(0, _TCH, _body, 0)

    @pl.when(c == 0)
    def _():
        _run(xmu_hbm)

    @pl.when(c == 1)
    def _():
        _run(xls_hbm)

    plsc.subcore_barrier()

    npw = _NP // _NTILE

    @pl.when(c == 0)
    def _():
        pltpu.sync_copy(acc.at[pl.ds(s * npw, npw)], omu_hbm.at[pl.ds(s * npw, npw)])

    @pl.when(c == 1)
    def _():
        pltpu.sync_copy(acc.at[pl.ds(s * npw, npw)], ols_hbm.at[pl.ds(s * npw, npw)])


def _agg(xws_mu, xws_ls, row2d, col2d):
    fn = functools.partial(
        pl.kernel,
        out_type=(jax.ShapeDtypeStruct((_NP, _D), jnp.float32),
                  jax.ShapeDtypeStruct((_NP, _D), jnp.float32)),
        mesh=_sc_mesh(),
        scratch_types=[
            pltpu.VMEM((_TCH, _CHUNK), jnp.int32),
            pltpu.VMEM((_TCH, _CHUNK), jnp.int32),
            pltpu.VMEM((_CHUNK, _D), jnp.float32),
            pltpu.VMEM_SHARED((_NP, _D), jnp.float32),
            pltpu.SemaphoreType.DMA,
        ],
    )(_agg_body)
    return fn(xws_mu, xws_ls, row2d, col2d)


def _fin_body(d_ref, amu_ref, als_ref, xmu_ref, xls_ref, eps_ref,
              bmu_ref, bls_ref, z_ref):
    dis = d_ref[...]
    mu = dis * (amu_ref[...] + xmu_ref[...]) + bmu_ref[...][:1]
    ls = dis * (als_ref[...] + xls_ref[...]) + bls_ref[...][:1]
    ls = jnp.minimum(ls, 10.0)
    z_ref[...] = mu + eps_ref[...] * jnp.exp(ls)


def _fin(dis_wide, acc_mu, acc_ls, xws_mu, xws_ls, eps_pad, b_mu8, b_ls8):
    br = 1024
    blk = pl.BlockSpec((br, _D), lambda i: (i, 0))
    bblk = pl.BlockSpec((8, _D), lambda i: (0, 0))
    return pl.pallas_call(
        _fin_body,
        grid=(_NP // br,),
        in_specs=[blk, blk, blk, blk, blk, blk, bblk, bblk],
        out_specs=blk,
        out_shape=jax.ShapeDtypeStruct((_NP, _D), jnp.float32),
    )(dis_wide, acc_mu, acc_ls, xws_mu, xws_ls, eps_pad, b_mu8, b_ls8)


def kernel(x, edge_index, eps, W_mu, b_mu, W_ls, b_ls):
    ei = edge_index.astype(jnp.int32)
    pad = _EP - _E
    row = jnp.concatenate([ei[0], jnp.full((pad,), _N, jnp.int32)])
    col = jnp.concatenate([ei[1], jnp.full((pad,), _N, jnp.int32)])
    row2d = row.reshape(_NCHUNK, _CHUNK)
    col2d = col.reshape(_NCHUNK, _CHUNK)
    x_pad = jnp.pad(x, ((0, _NP - _N), (0, 0)))
    eps_pad = jnp.pad(eps, ((0, _NP - _N), (0, 0)))
    b_mu8 = jnp.broadcast_to(b_mu[None, :], (8, _D))
    b_ls8 = jnp.broadcast_to(b_ls[None, :], (8, _D))

    dis_wide = _deg(col2d)
    xws_mu, xws_ls = _mm(x_pad, dis_wide, W_mu, W_ls)
    acc_mu, acc_ls = _agg(xws_mu, xws_ls, row2d, col2d)
    z = _fin(dis_wide, acc_mu, acc_ls, xws_mu, xws_ls, eps_pad, b_mu8, b_ls8)
    return z[:_N]


# speedup vs baseline: 8.6245x; 8.6245x over previous
import functools

import jax
import jax.numpy as jnp
from jax import lax
from jax.experimental import pallas as pl
from jax.experimental.pallas import tpu as pltpu
from jax.experimental.pallas import tpu_sc as plsc

_N = 10000
_D = 128
_E = 320000
_CHUNK = 128
_NSC = 2
_NTILE = 16
_NW = _NSC * _NTILE
_NP = 10240
_NCHUNK = -(-_E // _CHUNK)
_NCHUNK = -(-_NCHUNK // (_NTILE * 8)) * (_NTILE * 8)
_EP = _NCHUNK * _CHUNK
_TCH = _NCHUNK // _NTILE
_NODES_PER_W = _NP // _NW
_NQ = 4
_DQ = _D // _NQ
_BR = 1024


def _sc_mesh():
    return plsc.VectorSubcoreMesh(core_axis_name="c", subcore_axis_name="s")


def _deg_body(col_hbm, deg_hbm, col_v, ones_v, zbuf, rdbuf, acc, sem):
    c = lax.axis_index("c")
    s = lax.axis_index("s")
    w = c * _NTILE + s
    npt = _NP // _NTILE

    pltpu.sync_copy(col_hbm.at[pl.ds(s * _TCH, _TCH)], col_v)

    zeros16 = jnp.zeros((16,), jnp.float32)
    ones16 = jnp.ones((16,), jnp.float32)
    for m in range(_CHUNK // 16):
        ones_v[pl.ds(m * 16, 16)] = ones16
    for m in range(npt // 16):
        zbuf[pl.ds(m * 16, 16)] = zeros16
    pltpu.sync_copy(zbuf, acc.at[pl.ds(s * npt, npt)])
    plsc.subcore_barrier()

    def _hb(k, carry):
        pltpu.sync_copy(ones_v, acc.at[col_v.at[k]], add=True)
        return carry

    lax.fori_loop(0, _TCH, _hb, 0)
    plsc.subcore_barrier()

    sl = pl.ds(w * _NODES_PER_W, _NODES_PER_W)
    pltpu.sync_copy(acc.at[sl], rdbuf)
    pltpu.sync_copy(rdbuf, deg_hbm.at[sl])


def _deg(col2d):
    fn = functools.partial(
        pl.kernel,
        out_type=jax.ShapeDtypeStruct((_NP,), jnp.float32),
        mesh=_sc_mesh(),
        scratch_types=[
            pltpu.VMEM((_TCH, _CHUNK), jnp.int32),
            pltpu.VMEM((_CHUNK,), jnp.float32),
            pltpu.VMEM((_NP // _NTILE,), jnp.float32),
            pltpu.VMEM((_NODES_PER_W,), jnp.float32),
            pltpu.VMEM_SHARED((_NP,), jnp.float32),
            pltpu.SemaphoreType.DMA,
        ],
    )(_deg_body)
    return fn(col2d)


def _mm_body(x_ref, deg_ref, wm_ref, wl_ref, *out_refs):
    qrefs = out_refs[:2 * _NQ]
    od_ref = out_refs[2 * _NQ]
    dis8 = lax.rsqrt(deg_ref[...] + 1.0)
    ri = lax.broadcasted_iota(jnp.int32, (_BR, 8), 0) // _D
    ci = lax.broadcasted_iota(jnp.int32, (_BR, 8), 1)
    P = (ri == ci).astype(jnp.float32)
    T = jnp.dot(P, dis8, preferred_element_type=jnp.float32)
    li = lax.broadcasted_iota(jnp.int32, (_BR, _D), 1)
    rmod = lax.broadcasted_iota(jnp.int32, (_BR, _D), 0) % _D
    Tm = jnp.where(li == rmod, T, 0.0)
    dis_b = jnp.dot(Tm, jnp.ones((_D, _D), jnp.float32),
                    preferred_element_type=jnp.float32)
    od_ref[...] = dis_b
    xs = x_ref[...] * dis_b
    xwm = jnp.dot(xs, wm_ref[...], preferred_element_type=jnp.float32)
    xwl = jnp.dot(xs, wl_ref[...], preferred_element_type=jnp.float32)
    for q in range(_NQ):
        qrefs[q][...] = xwm[:, q * _DQ:(q + 1) * _DQ]
        qrefs[_NQ + q][...] = xwl[:, q * _DQ:(q + 1) * _DQ]


def _mm(x_pad, deg2d, W_mu, W_ls):
    blk = pl.BlockSpec((_BR, _D), lambda i: (i, 0))
    qblk = pl.BlockSpec((_BR, _DQ), lambda i: (i, 0))
    return pl.pallas_call(
        _mm_body,
        grid=(_NP // _BR,),
        in_specs=[
            blk,
            pl.BlockSpec((8, _D), lambda i: (i, 0)),
            pl.BlockSpec((_D, _D), lambda i: (0, 0)),
            pl.BlockSpec((_D, _D), lambda i: (0, 0)),
        ],
        out_specs=[qblk] * (2 * _NQ) + [blk],
        out_shape=[jax.ShapeDtypeStruct((_NP, _DQ), jnp.float32)] * (2 * _NQ)
        + [jax.ShapeDtypeStruct((_NP, _D), jnp.float32)],
    )(x_pad, deg2d, W_mu, W_ls)


def _agg_body(*refs):
    xq = refs[:2 * _NQ]
    row_hbm, col_hbm = refs[2 * _NQ], refs[2 * _NQ + 1]
    oq = refs[2 * _NQ + 2:4 * _NQ + 2]
    row_v, col_v, buf, zsrc, obuf, acc, sem = refs[4 * _NQ + 2:]

    c = lax.axis_index("c")
    s = lax.axis_index("s")
    npw = _NP // _NTILE

    pltpu.sync_copy(row_hbm.at[pl.ds(s * _TCH, _TCH)], row_v)
    pltpu.sync_copy(col_hbm.at[pl.ds(s * _TCH, _TCH)], col_v)

    zeros16 = jnp.zeros((16,), jnp.float32)

    def _zb(i, carry):
        for m in range(_DQ // 16):
            zsrc[i, pl.ds(m * 16, 16)] = zeros16
        return carry

    lax.fori_loop(0, _CHUNK, _zb, 0)
    nblk = npw // _CHUNK

    def _zero_acc():
        for j in range(nblk):
            pltpu.sync_copy(zsrc, acc.at[pl.ds((s * nblk + j) * _CHUNK, _CHUNK)])

    _zero_acc()
    plsc.subcore_barrier()

    def _pass(x_hbm, out_hbm, rezero):
        def _body(k, carry):
            pltpu.async_copy(x_hbm.at[row_v.at[k]], buf, sem).wait()
            pltpu.sync_copy(buf, acc.at[col_v.at[k]], add=True)
            return carry

        lax.fori_loop(0, _TCH, _body, 0)
        plsc.subcore_barrier()
        sl = pl.ds(s * npw, npw)
        pltpu.sync_copy(acc.at[sl], obuf)
        pltpu.sync_copy(obuf, out_hbm.at[sl])
        if rezero:
            _zero_acc()
        plsc.subcore_barrier()

    @pl.when(c == 0)
    def _():
        for q in range(_NQ):
            _pass(xq[q], oq[q], q < _NQ - 1)

    @pl.when(c == 1)
    def _():
        for q in range(_NQ):
            _pass(xq[_NQ + q], oq[_NQ + q], q < _NQ - 1)


def _agg(xqs, row2d, col2d):
    fn = functools.partial(
        pl.kernel,
        out_type=tuple(jax.ShapeDtypeStruct((_NP, _DQ), jnp.float32)
                       for _ in range(2 * _NQ)),
        mesh=_sc_mesh(),
        scratch_types=[
            pltpu.VMEM((_TCH, _CHUNK), jnp.int32),
            pltpu.VMEM((_TCH, _CHUNK), jnp.int32),
            pltpu.VMEM((_CHUNK, _DQ), jnp.float32),
            pltpu.VMEM((_CHUNK, _DQ), jnp.float32),
            pltpu.VMEM((_NP // _NTILE, _DQ), jnp.float32),
            pltpu.VMEM_SHARED((_NP, _DQ), jnp.float32),
            pltpu.SemaphoreType.DMA,
        ],
        compiler_params=pltpu.CompilerParams(use_tc_tiling_on_sc=False),
    )(_agg_body)
    return fn(*xqs, row2d, col2d)


def _fin_body(*refs):
    d_ref = refs[0]
    aq = refs[1:2 * _NQ + 1]
    xq = refs[2 * _NQ + 1:4 * _NQ + 1]
    eps_ref, bmu_ref, bls_ref, z_ref = refs[4 * _NQ + 1:]
    dis = d_ref[...]
    acc_mu = jnp.concatenate([aq[q][...] for q in range(_NQ)], axis=1)
    acc_ls = jnp.concatenate([aq[_NQ + q][...] for q in range(_NQ)], axis=1)
    xws_mu = jnp.concatenate([xq[q][...] for q in range(_NQ)], axis=1)
    xws_ls = jnp.concatenate([xq[_NQ + q][...] for q in range(_NQ)], axis=1)
    mu = dis * (acc_mu + xws_mu) + bmu_ref[...][:1]
    ls = dis * (acc_ls + xws_ls) + bls_ref[...][:1]
    ls = jnp.minimum(ls, 10.0)
    z_ref[...] = mu + eps_ref[...] * jnp.exp(ls)


def _fin(dis_wide, accs, xqs, eps_pad, b_mu8, b_ls8):
    blk = pl.BlockSpec((_BR, _D), lambda i: (i, 0))
    qblk = pl.BlockSpec((_BR, _DQ), lambda i: (i, 0))
    bblk = pl.BlockSpec((8, _D), lambda i: (0, 0))
    return pl.pallas_call(
        _fin_body,
        grid=(_NP // _BR,),
        in_specs=[blk] + [qblk] * (4 * _NQ) + [blk, bblk, bblk],
        out_specs=blk,
        out_shape=jax.ShapeDtypeStruct((_NP, _D), jnp.float32),
    )(dis_wide, *accs, *xqs, eps_pad, b_mu8, b_ls8)


def kernel(x, edge_index, eps, W_mu, b_mu, W_ls, b_ls):
    ei = edge_index.astype(jnp.int32)
    pad = _EP - _E
    row = jnp.concatenate([ei[0], jnp.full((pad,), _N, jnp.int32)])
    col = jnp.concatenate([ei[1], jnp.full((pad,), _N, jnp.int32)])
    row2d = row.reshape(_NCHUNK, _CHUNK)
    col2d = col.reshape(_NCHUNK, _CHUNK)
    x_pad = jnp.pad(x, ((0, _NP - _N), (0, 0)))
    eps_pad = jnp.pad(eps, ((0, _NP - _N), (0, 0)))
    b_mu8 = jnp.broadcast_to(b_mu[None, :], (8, _D))
    b_ls8 = jnp.broadcast_to(b_ls[None, :], (8, _D))

    deg1d = _deg(col2d)
    deg2d = deg1d.reshape(_NP // _D, _D)
    outs = _mm(x_pad, deg2d, W_mu, W_ls)
    xqs, dis_wide = outs[:2 * _NQ], outs[2 * _NQ]
    accs = _agg(xqs, row2d, col2d)
    z = _fin(dis_wide, accs, xqs, eps_pad, b_mu8, b_ls8)
    return z[:_N]


# speedup vs baseline: 12.9686x; 1.5037x over previous
import functools

import jax
import jax.numpy as jnp
from jax import lax
from jax.experimental import pallas as pl
from jax.experimental.pallas import tpu as pltpu
from jax.experimental.pallas import tpu_sc as plsc

_N = 10000
_D = 128
_E = 320000
_CHUNK = 128
_NSC = 2
_NTILE = 16
_NW = _NSC * _NTILE
_NP = 10240
_NCHUNK = -(-_E // _CHUNK)
_NCHUNK = -(-_NCHUNK // (_NTILE * 8)) * (_NTILE * 8)
_EP = _NCHUNK * _CHUNK
_TCH = _NCHUNK // _NTILE
_NODES_PER_W = _NP // _NW
_NQ = 4
_DQ = _D // _NQ
_BR = 1024


def _sc_mesh():
    return plsc.VectorSubcoreMesh(core_axis_name="c", subcore_axis_name="s")


def _deg_body(col_hbm, deg_hbm, col_v, ones_v, zbuf, rdbuf, acc, sem):
    c = lax.axis_index("c")
    s = lax.axis_index("s")
    w = c * _NTILE + s
    npt = _NP // _NTILE

    pltpu.sync_copy(col_hbm.at[pl.ds(s * _TCH, _TCH)], col_v)

    zeros16 = jnp.zeros((16,), jnp.float32)
    ones16 = jnp.ones((16,), jnp.float32)
    for m in range(_CHUNK // 16):
        ones_v[pl.ds(m * 16, 16)] = ones16
    for m in range(npt // 16):
        zbuf[pl.ds(m * 16, 16)] = zeros16
    pltpu.sync_copy(zbuf, acc.at[pl.ds(s * npt, npt)])
    plsc.subcore_barrier()

    def _hb(k, carry):
        pltpu.sync_copy(ones_v, acc.at[col_v.at[k]], add=True)
        return carry

    lax.fori_loop(0, _TCH, _hb, 0)
    plsc.subcore_barrier()

    sl = pl.ds(w * _NODES_PER_W, _NODES_PER_W)
    pltpu.sync_copy(acc.at[sl], rdbuf)
    pltpu.sync_copy(rdbuf, deg_hbm.at[sl])


def _deg(col2d):
    fn = functools.partial(
        pl.kernel,
        out_type=jax.ShapeDtypeStruct((_NP,), jnp.float32),
        mesh=_sc_mesh(),
        scratch_types=[
            pltpu.VMEM((_TCH, _CHUNK), jnp.int32),
            pltpu.VMEM((_CHUNK,), jnp.float32),
            pltpu.VMEM((_NP // _NTILE,), jnp.float32),
            pltpu.VMEM((_NODES_PER_W,), jnp.float32),
            pltpu.VMEM_SHARED((_NP,), jnp.float32),
            pltpu.SemaphoreType.DMA,
        ],
    )(_deg_body)
    return fn(col2d)


def _mm_body(x_ref, deg_ref, wm_ref, wl_ref, *out_refs):
    qrefs = out_refs[:2 * _NQ]
    od_ref = out_refs[2 * _NQ]
    dis8 = lax.rsqrt(deg_ref[...] + 1.0)
    ri = lax.broadcasted_iota(jnp.int32, (_BR, 8), 0) // _D
    ci = lax.broadcasted_iota(jnp.int32, (_BR, 8), 1)
    P = (ri == ci).astype(jnp.float32)
    T = jnp.dot(P, dis8, preferred_element_type=jnp.float32)
    li = lax.broadcasted_iota(jnp.int32, (_BR, _D), 1)
    rmod = lax.broadcasted_iota(jnp.int32, (_BR, _D), 0) % _D
    Tm = jnp.where(li == rmod, T, 0.0)
    dis_b = jnp.dot(Tm, jnp.ones((_D, _D), jnp.float32),
                    preferred_element_type=jnp.float32)
    od_ref[...] = dis_b
    xs = x_ref[...] * dis_b
    xwm = jnp.dot(xs, wm_ref[...], preferred_element_type=jnp.float32)
    xwl = jnp.dot(xs, wl_ref[...], preferred_element_type=jnp.float32)
    for q in range(_NQ):
        qrefs[q][...] = xwm[:, q * _DQ:(q + 1) * _DQ]
        qrefs[_NQ + q][...] = xwl[:, q * _DQ:(q + 1) * _DQ]


def _mm(x_pad, deg2d, W_mu, W_ls):
    blk = pl.BlockSpec((_BR, _D), lambda i: (i, 0))
    qblk = pl.BlockSpec((_BR, _DQ), lambda i: (i, 0))
    return pl.pallas_call(
        _mm_body,
        grid=(_NP // _BR,),
        in_specs=[
            blk,
            pl.BlockSpec((8, _D), lambda i: (i, 0)),
            pl.BlockSpec((_D, _D), lambda i: (0, 0)),
            pl.BlockSpec((_D, _D), lambda i: (0, 0)),
        ],
        out_specs=[qblk] * (2 * _NQ) + [blk],
        out_shape=[jax.ShapeDtypeStruct((_NP, _DQ), jnp.float32)] * (2 * _NQ)
        + [jax.ShapeDtypeStruct((_NP, _D), jnp.float32)],
    )(x_pad, deg2d, W_mu, W_ls)


_NB = 4


def _agg_body(*refs):
    xq = refs[:2 * _NQ]
    row_hbm, col_hbm = refs[2 * _NQ], refs[2 * _NQ + 1]
    oq = refs[2 * _NQ + 2:4 * _NQ + 2]
    row_v, col_v = refs[4 * _NQ + 2], refs[4 * _NQ + 3]
    bufs = refs[4 * _NQ + 4:4 * _NQ + 4 + _NB]
    zsrc, obuf, acc = refs[4 * _NQ + 4 + _NB:4 * _NQ + 7 + _NB]
    sems = refs[4 * _NQ + 7 + _NB:]

    c = lax.axis_index("c")
    s = lax.axis_index("s")
    npw = _NP // _NTILE

    pltpu.sync_copy(row_hbm.at[pl.ds(s * _TCH, _TCH)], row_v)
    pltpu.sync_copy(col_hbm.at[pl.ds(s * _TCH, _TCH)], col_v)

    zeros16 = jnp.zeros((16,), jnp.float32)

    def _zb(i, carry):
        for m in range(_DQ // 16):
            zsrc[i, pl.ds(m * 16, 16)] = zeros16
        return carry

    lax.fori_loop(0, _CHUNK, _zb, 0)
    nblk = npw // _CHUNK

    def _zero_acc():
        for j in range(nblk):
            pltpu.sync_copy(zsrc, acc.at[pl.ds((s * nblk + j) * _CHUNK, _CHUNK)])

    _zero_acc()
    plsc.subcore_barrier()

    def _pass(x_hbm, out_hbm, rezero):
        for b in range(_NB):
            pltpu.async_copy(x_hbm.at[row_v.at[b]], bufs[b], sems[b])

        def _body(g, carry):
            for b in range(_NB):
                k = g * _NB + b
                pltpu.make_async_copy(
                    x_hbm.at[row_v.at[k]], bufs[b], sems[b]).wait()
                pltpu.sync_copy(bufs[b], acc.at[col_v.at[k]], add=True)
                nk = k + _NB

                @pl.when(nk < _TCH)
                def _():
                    pltpu.async_copy(x_hbm.at[row_v.at[nk]], bufs[b], sems[b])
            return carry

        lax.fori_loop(0, _TCH // _NB, _body, 0)
        plsc.subcore_barrier()
        sl = pl.ds(s * npw, npw)
        pltpu.sync_copy(acc.at[sl], obuf)
        pltpu.sync_copy(obuf, out_hbm.at[sl])
        if rezero:
            _zero_acc()
        plsc.subcore_barrier()

    @pl.when(c == 0)
    def _():
        for q in range(_NQ):
            _pass(xq[q], oq[q], q < _NQ - 1)

    @pl.when(c == 1)
    def _():
        for q in range(_NQ):
            _pass(xq[_NQ + q], oq[_NQ + q], q < _NQ - 1)


def _agg(xqs, row2d, col2d):
    fn = functools.partial(
        pl.kernel,
        out_type=tuple(jax.ShapeDtypeStruct((_NP, _DQ), jnp.float32)
                       for _ in range(2 * _NQ)),
        mesh=_sc_mesh(),
        scratch_types=[
            pltpu.VMEM((_TCH, _CHUNK), jnp.int32),
            pltpu.VMEM((_TCH, _CHUNK), jnp.int32),
        ] + [pltpu.VMEM((_CHUNK, _DQ), jnp.float32)] * _NB + [
            pltpu.VMEM((_CHUNK, _DQ), jnp.float32),
            pltpu.VMEM((_NP // _NTILE, _DQ), jnp.float32),
            pltpu.VMEM_SHARED((_NP, _DQ), jnp.float32),
        ] + [pltpu.SemaphoreType.DMA] * _NB,
        compiler_params=pltpu.CompilerParams(use_tc_tiling_on_sc=False),
    )(_agg_body)
    return fn(*xqs, row2d, col2d)


def _fin_body(*refs):
    d_ref = refs[0]
    aq = refs[1:2 * _NQ + 1]
    xq = refs[2 * _NQ + 1:4 * _NQ + 1]
    eps_ref, bmu_ref, bls_ref, z_ref = refs[4 * _NQ + 1:]
    dis = d_ref[...]
    acc_mu = jnp.concatenate([aq[q][...] for q in range(_NQ)], axis=1)
    acc_ls = jnp.concatenate([aq[_NQ + q][...] for q in range(_NQ)], axis=1)
    xws_mu = jnp.concatenate([xq[q][...] for q in range(_NQ)], axis=1)
    xws_ls = jnp.concatenate([xq[_NQ + q][...] for q in range(_NQ)], axis=1)
    mu = dis * (acc_mu + xws_mu) + bmu_ref[...][:1]
    ls = dis * (acc_ls + xws_ls) + bls_ref[...][:1]
    ls = jnp.minimum(ls, 10.0)
    z_ref[...] = mu + eps_ref[...] * jnp.exp(ls)


def _fin(dis_wide, accs, xqs, eps_pad, b_mu8, b_ls8):
    blk = pl.BlockSpec((_BR, _D), lambda i: (i, 0))
    qblk = pl.BlockSpec((_BR, _DQ), lambda i: (i, 0))
    bblk = pl.BlockSpec((8, _D), lambda i: (0, 0))
    return pl.pallas_call(
        _fin_body,
        grid=(_NP // _BR,),
        in_specs=[blk] + [qblk] * (4 * _NQ) + [blk, bblk, bblk],
        out_specs=blk,
        out_shape=jax.ShapeDtypeStruct((_NP, _D), jnp.float32),
    )(dis_wide, *accs, *xqs, eps_pad, b_mu8, b_ls8)


def kernel(x, edge_index, eps, W_mu, b_mu, W_ls, b_ls):
    ei = edge_index.astype(jnp.int32)
    pad = _EP - _E
    row = jnp.concatenate([ei[0], jnp.full((pad,), _N, jnp.int32)])
    col = jnp.concatenate([ei[1], jnp.full((pad,), _N, jnp.int32)])
    row2d = row.reshape(_NCHUNK, _CHUNK)
    col2d = col.reshape(_NCHUNK, _CHUNK)
    x_pad = jnp.pad(x, ((0, _NP - _N), (0, 0)))
    eps_pad = jnp.pad(eps, ((0, _NP - _N), (0, 0)))
    b_mu8 = jnp.broadcast_to(b_mu[None, :], (8, _D))
    b_ls8 = jnp.broadcast_to(b_ls[None, :], (8, _D))

    deg1d = _deg(col2d)
    deg2d = deg1d.reshape(_NP // _D, _D)
    outs = _mm(x_pad, deg2d, W_mu, W_ls)
    xqs, dis_wide = outs[:2 * _NQ], outs[2 * _NQ]
    accs = _agg(xqs, row2d, col2d)
    z = _fin(dis_wide, accs, xqs, eps_pad, b_mu8, b_ls8)
    return z[:_N]


# speedup vs baseline: 13.1740x; 1.0158x over previous
import functools

import jax
import jax.numpy as jnp
from jax import lax
from jax.experimental import pallas as pl
from jax.experimental.pallas import tpu as pltpu
from jax.experimental.pallas import tpu_sc as plsc

_N = 10000
_D = 128
_E = 320000
_CHUNK = 128
_NSC = 2
_NTILE = 16
_NW = _NSC * _NTILE
_NP = 10240
_NCHUNK = -(-_E // _CHUNK)
_NCHUNK = -(-_NCHUNK // (_NTILE * 8)) * (_NTILE * 8)
_EP = _NCHUNK * _CHUNK
_TCH = _NCHUNK // _NTILE
_NODES_PER_W = _NP // _NW
_NQ = 4
_DQ = _D // _NQ
_BR = 1024


def _sc_mesh():
    return plsc.VectorSubcoreMesh(core_axis_name="c", subcore_axis_name="s")


def _deg_body(col_hbm, deg_hbm, col_v, ones_v, zbuf, rdbuf, acc, sem):
    c = lax.axis_index("c")
    s = lax.axis_index("s")
    w = c * _NTILE + s
    npt = _NP // _NTILE

    pltpu.sync_copy(col_hbm.at[pl.ds(s * _TCH, _TCH)], col_v)

    zeros16 = jnp.zeros((16,), jnp.float32)
    ones16 = jnp.ones((16,), jnp.float32)
    for m in range(_CHUNK // 16):
        ones_v[pl.ds(m * 16, 16)] = ones16
    for m in range(npt // 16):
        zbuf[pl.ds(m * 16, 16)] = zeros16
    pltpu.sync_copy(zbuf, acc.at[pl.ds(s * npt, npt)])
    plsc.subcore_barrier()

    def _hb(k, carry):
        pltpu.sync_copy(ones_v, acc.at[col_v.at[k]], add=True)
        return carry

    lax.fori_loop(0, _TCH, _hb, 0)
    plsc.subcore_barrier()

    sl = pl.ds(w * _NODES_PER_W, _NODES_PER_W)
    pltpu.sync_copy(acc.at[sl], rdbuf)
    pltpu.sync_copy(rdbuf, deg_hbm.at[sl])


def _deg(col2d):
    fn = functools.partial(
        pl.kernel,
        out_type=jax.ShapeDtypeStruct((_NP,), jnp.float32),
        mesh=_sc_mesh(),
        scratch_types=[
            pltpu.VMEM((_TCH, _CHUNK), jnp.int32),
            pltpu.VMEM((_CHUNK,), jnp.float32),
            pltpu.VMEM((_NP // _NTILE,), jnp.float32),
            pltpu.VMEM((_NODES_PER_W,), jnp.float32),
            pltpu.VMEM_SHARED((_NP,), jnp.float32),
            pltpu.SemaphoreType.DMA,
        ],
    )(_deg_body)
    return fn(col2d)


def _mm_body(x_ref, deg_ref, wm_ref, wl_ref, *out_refs):
    qrefs = out_refs[:2 * _NQ]
    od_ref = out_refs[2 * _NQ]
    dis8 = lax.rsqrt(deg_ref[...] + 1.0)
    ri = lax.broadcasted_iota(jnp.int32, (_BR, 8), 0) // _D
    ci = lax.broadcasted_iota(jnp.int32, (_BR, 8), 1)
    P = (ri == ci).astype(jnp.float32)
    T = jnp.dot(P, dis8, preferred_element_type=jnp.float32)
    li = lax.broadcasted_iota(jnp.int32, (_BR, _D), 1)
    rmod = lax.broadcasted_iota(jnp.int32, (_BR, _D), 0) % _D
    Tm = jnp.where(li == rmod, T, 0.0)
    dis_b = jnp.dot(Tm, jnp.ones((_D, _D), jnp.float32),
                    preferred_element_type=jnp.float32)
    od_ref[...] = dis_b
    xs = x_ref[...] * dis_b
    xwm = jnp.dot(xs, wm_ref[...], preferred_element_type=jnp.float32)
    xwl = jnp.dot(xs, wl_ref[...], preferred_element_type=jnp.float32)
    for q in range(_NQ):
        qrefs[q][...] = xwm[:, q * _DQ:(q + 1) * _DQ]
        qrefs[_NQ + q][...] = xwl[:, q * _DQ:(q + 1) * _DQ]


def _mm(x_pad, deg2d, W_mu, W_ls):
    blk = pl.BlockSpec((_BR, _D), lambda i: (i, 0))
    qblk = pl.BlockSpec((_BR, _DQ), lambda i: (i, 0))
    return pl.pallas_call(
        _mm_body,
        grid=(_NP // _BR,),
        in_specs=[
            blk,
            pl.BlockSpec((8, _D), lambda i: (i, 0)),
            pl.BlockSpec((_D, _D), lambda i: (0, 0)),
            pl.BlockSpec((_D, _D), lambda i: (0, 0)),
        ],
        out_specs=[qblk] * (2 * _NQ) + [blk],
        out_shape=[jax.ShapeDtypeStruct((_NP, _DQ), jnp.float32)] * (2 * _NQ)
        + [jax.ShapeDtypeStruct((_NP, _D), jnp.float32)],
    )(x_pad, deg2d, W_mu, W_ls)


_NB = 8


def _agg_body(*refs):
    xq = refs[:2 * _NQ]
    row_hbm, col_hbm = refs[2 * _NQ], refs[2 * _NQ + 1]
    oq = refs[2 * _NQ + 2:4 * _NQ + 2]
    row_v, col_v = refs[4 * _NQ + 2], refs[4 * _NQ + 3]
    bufs = refs[4 * _NQ + 4:4 * _NQ + 4 + _NB]
    zsrc, obuf, acc = refs[4 * _NQ + 4 + _NB:4 * _NQ + 7 + _NB]
    sems = refs[4 * _NQ + 7 + _NB:4 * _NQ + 7 + 2 * _NB]
    asems = refs[4 * _NQ + 7 + 2 * _NB:]

    c = lax.axis_index("c")
    s = lax.axis_index("s")
    npw = _NP // _NTILE

    pltpu.sync_copy(row_hbm.at[pl.ds(s * _TCH, _TCH)], row_v)
    pltpu.sync_copy(col_hbm.at[pl.ds(s * _TCH, _TCH)], col_v)

    zeros16 = jnp.zeros((16,), jnp.float32)

    def _zb(i, carry):
        for m in range(_DQ // 16):
            zsrc[i, pl.ds(m * 16, 16)] = zeros16
        return carry

    lax.fori_loop(0, _CHUNK, _zb, 0)
    nblk = npw // _CHUNK

    def _zero_acc():
        for j in range(nblk):
            pltpu.sync_copy(zsrc, acc.at[pl.ds((s * nblk + j) * _CHUNK, _CHUNK)])

    _zero_acc()
    plsc.subcore_barrier()

    def _pass(x_hbm, out_hbm, rezero):
        for b in range(_NB):
            pltpu.async_copy(x_hbm.at[row_v.at[b]], bufs[b], sems[b])

        def _body(g, carry):
            for b in range(_NB):
                k = g * _NB + b
                pltpu.make_async_copy(
                    x_hbm.at[row_v.at[k]], bufs[b], sems[b]).wait()
                pltpu.async_copy(bufs[b], acc.at[col_v.at[k]], asems[b],
                                 add=True)
            for b in range(_NB):
                k = g * _NB + b
                pltpu.make_async_copy(bufs[b], acc.at[col_v.at[k]],
                                      asems[b]).wait()
                nk = k + _NB

                @pl.when(nk < _TCH)
                def _():
                    pltpu.async_copy(x_hbm.at[row_v.at[nk]], bufs[b], sems[b])
            return carry

        lax.fori_loop(0, _TCH // _NB, _body, 0)
        plsc.subcore_barrier()
        sl = pl.ds(s * npw, npw)
        pltpu.sync_copy(acc.at[sl], obuf)
        pltpu.sync_copy(obuf, out_hbm.at[sl])
        if rezero:
            _zero_acc()
        plsc.subcore_barrier()

    @pl.when(c == 0)
    def _():
        for q in range(_NQ):
            _pass(xq[q], oq[q], q < _NQ - 1)

    @pl.when(c == 1)
    def _():
        for q in range(_NQ):
            _pass(xq[_NQ + q], oq[_NQ + q], q < _NQ - 1)


def _agg(xqs, row2d, col2d):
    fn = functools.partial(
        pl.kernel,
        out_type=tuple(jax.ShapeDtypeStruct((_NP, _DQ), jnp.float32)
                       for _ in range(2 * _NQ)),
        mesh=_sc_mesh(),
        scratch_types=[
            pltpu.VMEM((_TCH, _CHUNK), jnp.int32),
            pltpu.VMEM((_TCH, _CHUNK), jnp.int32),
        ] + [pltpu.VMEM((_CHUNK, _DQ), jnp.float32)] * _NB + [
            pltpu.VMEM((_CHUNK, _DQ), jnp.float32),
            pltpu.VMEM((_NP // _NTILE, _DQ), jnp.float32),
            pltpu.VMEM_SHARED((_NP, _DQ), jnp.float32),
        ] + [pltpu.SemaphoreType.DMA] * (2 * _NB),
        compiler_params=pltpu.CompilerParams(use_tc_tiling_on_sc=False),
    )(_agg_body)
    return fn(*xqs, row2d, col2d)


def _fin_body(*refs):
    d_ref = refs[0]
    aq = refs[1:2 * _NQ + 1]
    xq = refs[2 * _NQ + 1:4 * _NQ + 1]
    eps_ref, bmu_ref, bls_ref, z_ref = refs[4 * _NQ + 1:]
    dis = d_ref[...]
    acc_mu = jnp.concatenate([aq[q][...] for q in range(_NQ)], axis=1)
    acc_ls = jnp.concatenate([aq[_NQ + q][...] for q in range(_NQ)], axis=1)
    xws_mu = jnp.concatenate([xq[q][...] for q in range(_NQ)], axis=1)
    xws_ls = jnp.concatenate([xq[_NQ + q][...] for q in range(_NQ)], axis=1)
    mu = dis * (acc_mu + xws_mu) + bmu_ref[...][:1]
    ls = dis * (acc_ls + xws_ls) + bls_ref[...][:1]
    ls = jnp.minimum(ls, 10.0)
    z_ref[...] = mu + eps_ref[...] * jnp.exp(ls)


def _fin(dis_wide, accs, xqs, eps_pad, b_mu8, b_ls8):
    blk = pl.BlockSpec((_BR, _D), lambda i: (i, 0))
    qblk = pl.BlockSpec((_BR, _DQ), lambda i: (i, 0))
    bblk = pl.BlockSpec((8, _D), lambda i: (0, 0))
    return pl.pallas_call(
        _fin_body,
        grid=(_NP // _BR,),
        in_specs=[blk] + [qblk] * (4 * _NQ) + [blk, bblk, bblk],
        out_specs=blk,
        out_shape=jax.ShapeDtypeStruct((_NP, _D), jnp.float32),
    )(dis_wide, *accs, *xqs, eps_pad, b_mu8, b_ls8)


def kernel(x, edge_index, eps, W_mu, b_mu, W_ls, b_ls):
    ei = edge_index.astype(jnp.int32)
    pad = _EP - _E
    row = jnp.concatenate([ei[0], jnp.full((pad,), _N, jnp.int32)])
    col = jnp.concatenate([ei[1], jnp.full((pad,), _N, jnp.int32)])
    row2d = row.reshape(_NCHUNK, _CHUNK)
    col2d = col.reshape(_NCHUNK, _CHUNK)
    x_pad = jnp.pad(x, ((0, _NP - _N), (0, 0)))
    eps_pad = jnp.pad(eps, ((0, _NP - _N), (0, 0)))
    b_mu8 = jnp.broadcast_to(b_mu[None, :], (8, _D))
    b_ls8 = jnp.broadcast_to(b_ls[None, :], (8, _D))

    deg1d = _deg(col2d)
    deg2d = deg1d.reshape(_NP // _D, _D)
    outs = _mm(x_pad, deg2d, W_mu, W_ls)
    xqs, dis_wide = outs[:2 * _NQ], outs[2 * _NQ]
    accs = _agg(xqs, row2d, col2d)
    z = _fin(dis_wide, accs, xqs, eps_pad, b_mu8, b_ls8)
    return z[:_N]


# speedup vs baseline: 13.2625x; 1.0067x over previous
import functools

import jax
import jax.numpy as jnp
from jax import lax
from jax.experimental import pallas as pl
from jax.experimental.pallas import tpu as pltpu
from jax.experimental.pallas import tpu_sc as plsc

_N = 10000
_D = 128
_E = 320000
_CHUNK = 128
_NSC = 2
_NTILE = 16
_NW = _NSC * _NTILE
_NP = 10240
_NCHUNK = -(-_E // _CHUNK)
_NCHUNK = -(-_NCHUNK // (_NTILE * 8)) * (_NTILE * 8)
_EP = _NCHUNK * _CHUNK
_TCH = _NCHUNK // _NTILE
_NODES_PER_W = _NP // _NW
_NQ = 2
_DQ = _D // _NQ
_BR = 1024


def _sc_mesh():
    return plsc.VectorSubcoreMesh(core_axis_name="c", subcore_axis_name="s")


def _deg_body(col_hbm, deg_hbm, col_v, ones_v, zbuf, rdbuf, acc, sem):
    c = lax.axis_index("c")
    s = lax.axis_index("s")
    w = c * _NTILE + s
    npt = _NP // _NTILE

    pltpu.sync_copy(col_hbm.at[pl.ds(s * _TCH, _TCH)], col_v)

    zeros16 = jnp.zeros((16,), jnp.float32)
    ones16 = jnp.ones((16,), jnp.float32)
    for m in range(_CHUNK // 16):
        ones_v[pl.ds(m * 16, 16)] = ones16
    for m in range(npt // 16):
        zbuf[pl.ds(m * 16, 16)] = zeros16
    pltpu.sync_copy(zbuf, acc.at[pl.ds(s * npt, npt)])
    plsc.subcore_barrier()

    def _hb(k, carry):
        pltpu.sync_copy(ones_v, acc.at[col_v.at[k]], add=True)
        return carry

    lax.fori_loop(0, _TCH, _hb, 0)
    plsc.subcore_barrier()

    sl = pl.ds(w * _NODES_PER_W, _NODES_PER_W)
    pltpu.sync_copy(acc.at[sl], rdbuf)
    pltpu.sync_copy(rdbuf, deg_hbm.at[sl])


def _deg(col2d):
    fn = functools.partial(
        pl.kernel,
        out_type=jax.ShapeDtypeStruct((_NP,), jnp.float32),
        mesh=_sc_mesh(),
        scratch_types=[
            pltpu.VMEM((_TCH, _CHUNK), jnp.int32),
            pltpu.VMEM((_CHUNK,), jnp.float32),
            pltpu.VMEM((_NP // _NTILE,), jnp.float32),
            pltpu.VMEM((_NODES_PER_W,), jnp.float32),
            pltpu.VMEM_SHARED((_NP,), jnp.float32),
            pltpu.SemaphoreType.DMA,
        ],
    )(_deg_body)
    return fn(col2d)


def _mm_body(x_ref, deg_ref, wm_ref, wl_ref, *out_refs):
    qrefs = out_refs[:2 * _NQ]
    od_ref = out_refs[2 * _NQ]
    dis8 = lax.rsqrt(deg_ref[...] + 1.0)
    ri = lax.broadcasted_iota(jnp.int32, (_BR, 8), 0) // _D
    ci = lax.broadcasted_iota(jnp.int32, (_BR, 8), 1)
    P = (ri == ci).astype(jnp.float32)
    T = jnp.dot(P, dis8, preferred_element_type=jnp.float32)
    li = lax.broadcasted_iota(jnp.int32, (_BR, _D), 1)
    rmod = lax.broadcasted_iota(jnp.int32, (_BR, _D), 0) % _D
    Tm = jnp.where(li == rmod, T, 0.0)
    dis_b = jnp.dot(Tm, jnp.ones((_D, _D), jnp.float32),
                    preferred_element_type=jnp.float32)
    od_ref[...] = dis_b
    xs = x_ref[...] * dis_b
    xwm = jnp.dot(xs, wm_ref[...], preferred_element_type=jnp.float32)
    xwl = jnp.dot(xs, wl_ref[...], preferred_element_type=jnp.float32)
    for q in range(_NQ):
        qrefs[q][...] = xwm[:, q * _DQ:(q + 1) * _DQ]
        qrefs[_NQ + q][...] = xwl[:, q * _DQ:(q + 1) * _DQ]


def _mm(x_pad, deg2d, W_mu, W_ls):
    blk = pl.BlockSpec((_BR, _D), lambda i: (i, 0))
    qblk = pl.BlockSpec((_BR, _DQ), lambda i: (i, 0))
    return pl.pallas_call(
        _mm_body,
        grid=(_NP // _BR,),
        in_specs=[
            blk,
            pl.BlockSpec((8, _D), lambda i: (i, 0)),
            pl.BlockSpec((_D, _D), lambda i: (0, 0)),
            pl.BlockSpec((_D, _D), lambda i: (0, 0)),
        ],
        out_specs=[qblk] * (2 * _NQ) + [blk],
        out_shape=[jax.ShapeDtypeStruct((_NP, _DQ), jnp.float32)] * (2 * _NQ)
        + [jax.ShapeDtypeStruct((_NP, _D), jnp.float32)],
    )(x_pad, deg2d, W_mu, W_ls)


_NB = 5


def _agg_body(*refs):
    xq = refs[:2 * _NQ]
    row_hbm, col_hbm = refs[2 * _NQ], refs[2 * _NQ + 1]
    oq = refs[2 * _NQ + 2:4 * _NQ + 2]
    row_v, col_v = refs[4 * _NQ + 2], refs[4 * _NQ + 3]
    bufs = refs[4 * _NQ + 4:4 * _NQ + 4 + _NB]
    acc = refs[4 * _NQ + 4 + _NB]
    sems = refs[4 * _NQ + 5 + _NB:4 * _NQ + 5 + 2 * _NB]
    asems = refs[4 * _NQ + 5 + 2 * _NB:]

    c = lax.axis_index("c")
    s = lax.axis_index("s")
    npw = _NP // _NTILE
    nblk = npw // _CHUNK

    pltpu.sync_copy(row_hbm.at[pl.ds(s * _TCH, _TCH)], row_v)
    pltpu.sync_copy(col_hbm.at[pl.ds(s * _TCH, _TCH)], col_v)

    zeros16 = jnp.zeros((16,), jnp.float32)

    def _zero_acc():
        def _zb(i, carry):
            for m in range(_DQ // 16):
                bufs[0][i, pl.ds(m * 16, 16)] = zeros16
            return carry

        lax.fori_loop(0, _CHUNK, _zb, 0)
        for j in range(nblk):
            pltpu.sync_copy(bufs[0],
                            acc.at[pl.ds((s * nblk + j) * _CHUNK, _CHUNK)])

    _zero_acc()
    plsc.subcore_barrier()

    def _pass(x_hbm, out_hbm, rezero):
        for b in range(_NB):
            pltpu.async_copy(x_hbm.at[row_v.at[b]], bufs[b], sems[b])

        def _body(g, carry):
            for b in range(_NB):
                k = g * _NB + b
                pltpu.make_async_copy(
                    x_hbm.at[row_v.at[k]], bufs[b], sems[b]).wait()
                pltpu.async_copy(bufs[b], acc.at[col_v.at[k]], asems[b],
                                 add=True)
            for b in range(_NB):
                k = g * _NB + b
                pltpu.make_async_copy(bufs[b], acc.at[col_v.at[k]],
                                      asems[b]).wait()
                nk = k + _NB

                @pl.when(nk < _TCH)
                def _():
                    pltpu.async_copy(x_hbm.at[row_v.at[nk]], bufs[b], sems[b])
            return carry

        lax.fori_loop(0, _TCH // _NB, _body, 0)
        plsc.subcore_barrier()
        for j in range(nblk):
            blk = pl.ds((s * nblk + j) * _CHUNK, _CHUNK)
            pltpu.sync_copy(acc.at[blk], bufs[j % _NB])
            pltpu.sync_copy(bufs[j % _NB], out_hbm.at[blk])
        if rezero:
            _zero_acc()
        plsc.subcore_barrier()

    @pl.when(c == 0)
    def _():
        for q in range(_NQ):
            _pass(xq[q], oq[q], q < _NQ - 1)

    @pl.when(c == 1)
    def _():
        for q in range(_NQ):
            _pass(xq[_NQ + q], oq[_NQ + q], q < _NQ - 1)


def _agg(xqs, row2d, col2d):
    fn = functools.partial(
        pl.kernel,
        out_type=tuple(jax.ShapeDtypeStruct((_NP, _DQ), jnp.float32)
                       for _ in range(2 * _NQ)),
        mesh=_sc_mesh(),
        scratch_types=[
            pltpu.VMEM((_TCH, _CHUNK), jnp.int32),
            pltpu.VMEM((_TCH, _CHUNK), jnp.int32),
        ] + [pltpu.VMEM((_CHUNK, _DQ), jnp.float32)] * _NB + [
            pltpu.VMEM_SHARED((_NP, _DQ), jnp.float32),
        ] + [pltpu.SemaphoreType.DMA] * (2 * _NB),
        compiler_params=pltpu.CompilerParams(use_tc_tiling_on_sc=False),
    )(_agg_body)
    return fn(*xqs, row2d, col2d)


def _fin_body(*refs):
    d_ref = refs[0]
    aq = refs[1:2 * _NQ + 1]
    xq = refs[2 * _NQ + 1:4 * _NQ + 1]
    eps_ref, bmu_ref, bls_ref, z_ref = refs[4 * _NQ + 1:]
    dis = d_ref[...]
    acc_mu = jnp.concatenate([aq[q][...] for q in range(_NQ)], axis=1)
    acc_ls = jnp.concatenate([aq[_NQ + q][...] for q in range(_NQ)], axis=1)
    xws_mu = jnp.concatenate([xq[q][...] for q in range(_NQ)], axis=1)
    xws_ls = jnp.concatenate([xq[_NQ + q][...] for q in range(_NQ)], axis=1)
    mu = dis * (acc_mu + xws_mu) + bmu_ref[...][:1]
    ls = dis * (acc_ls + xws_ls) + bls_ref[...][:1]
    ls = jnp.minimum(ls, 10.0)
    z_ref[...] = mu + eps_ref[...] * jnp.exp(ls)


def _fin(dis_wide, accs, xqs, eps_pad, b_mu8, b_ls8):
    blk = pl.BlockSpec((_BR, _D), lambda i: (i, 0))
    qblk = pl.BlockSpec((_BR, _DQ), lambda i: (i, 0))
    bblk = pl.BlockSpec((8, _D), lambda i: (0, 0))
    return pl.pallas_call(
        _fin_body,
        grid=(_NP // _BR,),
        in_specs=[blk] + [qblk] * (4 * _NQ) + [blk, bblk, bblk],
        out_specs=blk,
        out_shape=jax.ShapeDtypeStruct((_NP, _D), jnp.float32),
    )(dis_wide, *accs, *xqs, eps_pad, b_mu8, b_ls8)


def kernel(x, edge_index, eps, W_mu, b_mu, W_ls, b_ls):
    ei = edge_index.astype(jnp.int32)
    pad = _EP - _E
    row = jnp.concatenate([ei[0], jnp.full((pad,), _N, jnp.int32)])
    col = jnp.concatenate([ei[1], jnp.full((pad,), _N, jnp.int32)])
    row2d = row.reshape(_NCHUNK, _CHUNK)
    col2d = col.reshape(_NCHUNK, _CHUNK)
    x_pad = jnp.pad(x, ((0, _NP - _N), (0, 0)))
    eps_pad = jnp.pad(eps, ((0, _NP - _N), (0, 0)))
    b_mu8 = jnp.broadcast_to(b_mu[None, :], (8, _D))
    b_ls8 = jnp.broadcast_to(b_ls[None, :], (8, _D))

    deg1d = _deg(col2d)
    deg2d = deg1d.reshape(_NP // _D, _D)
    outs = _mm(x_pad, deg2d, W_mu, W_ls)
    xqs, dis_wide = outs[:2 * _NQ], outs[2 * _NQ]
    accs = _agg(xqs, row2d, col2d)
    z = _fin(dis_wide, accs, xqs, eps_pad, b_mu8, b_ls8)
    return z[:_N]


# speedup vs baseline: 13.2989x; 1.0027x over previous
import functools

import jax
import jax.numpy as jnp
from jax import lax
from jax.experimental import pallas as pl
from jax.experimental.pallas import tpu as pltpu
from jax.experimental.pallas import tpu_sc as plsc

_N = 10000
_D = 128
_E = 320000
_CHUNK = 128
_NSC = 2
_NTILE = 16
_NW = _NSC * _NTILE
_NP = 10240
_NCHUNK = -(-_E // _CHUNK)
_NCHUNK = -(-_NCHUNK // (_NTILE * 8)) * (_NTILE * 8)
_EP = _NCHUNK * _CHUNK
_TCH = _NCHUNK // _NTILE
_NODES_PER_W = _NP // _NW
_NQ = 1
_DQ = _D // _NQ
_BR = 1024


def _sc_mesh():
    return plsc.VectorSubcoreMesh(core_axis_name="c", subcore_axis_name="s")


def _deg_body(col_hbm, deg_hbm, col_v, ones_v, zbuf, rdbuf, acc, sem):
    c = lax.axis_index("c")
    s = lax.axis_index("s")
    w = c * _NTILE + s
    npt = _NP // _NTILE

    pltpu.sync_copy(col_hbm.at[pl.ds(s * _TCH, _TCH)], col_v)

    zeros16 = jnp.zeros((16,), jnp.float32)
    ones16 = jnp.ones((16,), jnp.float32)
    for m in range(_CHUNK // 16):
        ones_v[pl.ds(m * 16, 16)] = ones16
    for m in range(npt // 16):
        zbuf[pl.ds(m * 16, 16)] = zeros16
    pltpu.sync_copy(zbuf, acc.at[pl.ds(s * npt, npt)])
    plsc.subcore_barrier()

    def _hb(k, carry):
        pltpu.sync_copy(ones_v, acc.at[col_v.at[k]], add=True)
        return carry

    lax.fori_loop(0, _TCH, _hb, 0)
    plsc.subcore_barrier()

    sl = pl.ds(w * _NODES_PER_W, _NODES_PER_W)
    pltpu.sync_copy(acc.at[sl], rdbuf)
    pltpu.sync_copy(rdbuf, deg_hbm.at[sl])


def _deg(col2d):
    fn = functools.partial(
        pl.kernel,
        out_type=jax.ShapeDtypeStruct((_NP,), jnp.float32),
        mesh=_sc_mesh(),
        scratch_types=[
            pltpu.VMEM((_TCH, _CHUNK), jnp.int32),
            pltpu.VMEM((_CHUNK,), jnp.float32),
            pltpu.VMEM((_NP // _NTILE,), jnp.float32),
            pltpu.VMEM((_NODES_PER_W,), jnp.float32),
            pltpu.VMEM_SHARED((_NP,), jnp.float32),
            pltpu.SemaphoreType.DMA,
        ],
    )(_deg_body)
    return fn(col2d)


def _mm_body(x_ref, deg_ref, wm_ref, wl_ref, *out_refs):
    qrefs = out_refs[:2 * _NQ]
    od_ref = out_refs[2 * _NQ]
    dis8 = lax.rsqrt(deg_ref[...] + 1.0)
    ri = lax.broadcasted_iota(jnp.int32, (_BR, 8), 0) // _D
    ci = lax.broadcasted_iota(jnp.int32, (_BR, 8), 1)
    P = (ri == ci).astype(jnp.float32)
    T = jnp.dot(P, dis8, preferred_element_type=jnp.float32)
    li = lax.broadcasted_iota(jnp.int32, (_BR, _D), 1)
    rmod = lax.broadcasted_iota(jnp.int32, (_BR, _D), 0) % _D
    Tm = jnp.where(li == rmod, T, 0.0)
    dis_b = jnp.dot(Tm, jnp.ones((_D, _D), jnp.float32),
                    preferred_element_type=jnp.float32)
    od_ref[...] = dis_b
    xs = x_ref[...] * dis_b
    xwm = jnp.dot(xs, wm_ref[...], preferred_element_type=jnp.float32)
    xwl = jnp.dot(xs, wl_ref[...], preferred_element_type=jnp.float32)
    for q in range(_NQ):
        qrefs[q][...] = xwm[:, q * _DQ:(q + 1) * _DQ]
        qrefs[_NQ + q][...] = xwl[:, q * _DQ:(q + 1) * _DQ]


def _mm(x_pad, deg2d, W_mu, W_ls):
    blk = pl.BlockSpec((_BR, _D), lambda i: (i, 0))
    qblk = pl.BlockSpec((_BR, _DQ), lambda i: (i, 0))
    return pl.pallas_call(
        _mm_body,
        grid=(_NP // _BR,),
        in_specs=[
            blk,
            pl.BlockSpec((8, _D), lambda i: (i, 0)),
            pl.BlockSpec((_D, _D), lambda i: (0, 0)),
            pl.BlockSpec((_D, _D), lambda i: (0, 0)),
        ],
        out_specs=[qblk] * (2 * _NQ) + [blk],
        out_shape=[jax.ShapeDtypeStruct((_NP, _DQ), jnp.float32)] * (2 * _NQ)
        + [jax.ShapeDtypeStruct((_NP, _D), jnp.float32)],
    )(x_pad, deg2d, W_mu, W_ls)


_NB = 2
_IB = 20
_NIB = _TCH // _IB


def _agg_body(*refs):
    xq = refs[:2 * _NQ]
    row_hbm, col_hbm = refs[2 * _NQ], refs[2 * _NQ + 1]
    oq = refs[2 * _NQ + 2:4 * _NQ + 2]
    base = 4 * _NQ + 2
    rb = refs[base:base + 2]
    cb = refs[base + 2:base + 4]
    bufs = refs[base + 4:base + 4 + _NB]
    acc = refs[base + 4 + _NB]
    rsem = refs[base + 5 + _NB:base + 7 + _NB]
    csem = refs[base + 7 + _NB:base + 9 + _NB]
    sems = refs[base + 9 + _NB:base + 9 + 2 * _NB]
    asems = refs[base + 9 + 2 * _NB:]

    c = lax.axis_index("c")
    s = lax.axis_index("s")
    npw = _NP // _NTILE
    nblk = npw // _CHUNK

    def _idx_fetch(i, p):
        sl = pl.ds(s * _TCH + i * _IB, _IB)
        pltpu.async_copy(row_hbm.at[sl], rb[p], rsem[p])
        pltpu.async_copy(col_hbm.at[sl], cb[p], csem[p])

    def _idx_wait(i, p):
        sl = pl.ds(s * _TCH + i * _IB, _IB)
        pltpu.make_async_copy(row_hbm.at[sl], rb[p], rsem[p]).wait()
        pltpu.make_async_copy(col_hbm.at[sl], cb[p], csem[p]).wait()

    _idx_fetch(0, 0)

    zeros16 = jnp.zeros((16,), jnp.float32)

    def _zb(i, carry):
        for m in range(_DQ // 16):
            bufs[0][i, pl.ds(m * 16, 16)] = zeros16
        return carry

    lax.fori_loop(0, _CHUNK, _zb, 0)
    for j in range(nblk):
        pltpu.sync_copy(bufs[0], acc.at[pl.ds((s * nblk + j) * _CHUNK, _CHUNK)])
    plsc.subcore_barrier()

    def _pass(x_hbm, out_hbm):
        for i in range(_NIB):
            p = i & 1
            _idx_wait(i, p)
            if i + 1 < _NIB:
                _idx_fetch(i + 1, 1 - p)
            for b in range(_NB):
                pltpu.async_copy(x_hbm.at[rb[p].at[b]], bufs[b], sems[b])

            def _body(g, carry):
                for b in range(_NB):
                    k = g * _NB + b
                    pltpu.make_async_copy(
                        x_hbm.at[rb[p].at[k]], bufs[b], sems[b]).wait()
                    pltpu.async_copy(bufs[b], acc.at[cb[p].at[k]], asems[b],
                                     add=True)
                for b in range(_NB):
                    k = g * _NB + b
                    pltpu.make_async_copy(bufs[b], acc.at[cb[p].at[k]],
                                          asems[b]).wait()
                    nk = k + _NB

                    @pl.when(nk < _IB)
                    def _():
                        pltpu.async_copy(x_hbm.at[rb[p].at[nk]], bufs[b],
                                         sems[b])
                return carry

            lax.fori_loop(0, _IB // _NB, _body, 0)
        plsc.subcore_barrier()
        for j in range(nblk):
            blk = pl.ds((s * nblk + j) * _CHUNK, _CHUNK)
            pltpu.sync_copy(acc.at[blk], bufs[j % _NB])
            pltpu.sync_copy(bufs[j % _NB], out_hbm.at[blk])

    @pl.when(c == 0)
    def _():
        _pass(xq[0], oq[0])

    @pl.when(c == 1)
    def _():
        _pass(xq[1], oq[1])


def _agg(xqs, row2d, col2d):
    fn = functools.partial(
        pl.kernel,
        out_type=tuple(jax.ShapeDtypeStruct((_NP, _DQ), jnp.float32)
                       for _ in range(2 * _NQ)),
        mesh=_sc_mesh(),
        scratch_types=[
            pltpu.VMEM((_IB, _CHUNK), jnp.int32),
            pltpu.VMEM((_IB, _CHUNK), jnp.int32),
            pltpu.VMEM((_IB, _CHUNK), jnp.int32),
            pltpu.VMEM((_IB, _CHUNK), jnp.int32),
        ] + [pltpu.VMEM((_CHUNK, _DQ), jnp.float32)] * _NB + [
            pltpu.VMEM_SHARED((_NP, _DQ), jnp.float32),
        ] + [pltpu.SemaphoreType.DMA] * (4 + 2 * _NB),
        compiler_params=pltpu.CompilerParams(use_tc_tiling_on_sc=False),
    )(_agg_body)
    return fn(*xqs, row2d, col2d)


def _fin_body(*refs):
    d_ref = refs[0]
    aq = refs[1:2 * _NQ + 1]
    xq = refs[2 * _NQ + 1:4 * _NQ + 1]
    eps_ref, bmu_ref, bls_ref, z_ref = refs[4 * _NQ + 1:]
    dis = d_ref[...]
    acc_mu = jnp.concatenate([aq[q][...] for q in range(_NQ)], axis=1)
    acc_ls = jnp.concatenate([aq[_NQ + q][...] for q in range(_NQ)], axis=1)
    xws_mu = jnp.concatenate([xq[q][...] for q in range(_NQ)], axis=1)
    xws_ls = jnp.concatenate([xq[_NQ + q][...] for q in range(_NQ)], axis=1)
    mu = dis * (acc_mu + xws_mu) + bmu_ref[...][:1]
    ls = dis * (acc_ls + xws_ls) + bls_ref[...][:1]
    ls = jnp.minimum(ls, 10.0)
    z_ref[...] = mu + eps_ref[...] * jnp.exp(ls)


def _fin(dis_wide, accs, xqs, eps_pad, b_mu8, b_ls8):
    blk = pl.BlockSpec((_BR, _D), lambda i: (i, 0))
    qblk = pl.BlockSpec((_BR, _DQ), lambda i: (i, 0))
    bblk = pl.BlockSpec((8, _D), lambda i: (0, 0))
    return pl.pallas_call(
        _fin_body,
        grid=(_NP // _BR,),
        in_specs=[blk] + [qblk] * (4 * _NQ) + [blk, bblk, bblk],
        out_specs=blk,
        out_shape=jax.ShapeDtypeStruct((_NP, _D), jnp.float32),
    )(dis_wide, *accs, *xqs, eps_pad, b_mu8, b_ls8)


def kernel(x, edge_index, eps, W_mu, b_mu, W_ls, b_ls):
    ei = edge_index.astype(jnp.int32)
    pad = _EP - _E
    row = jnp.concatenate([ei[0], jnp.full((pad,), _N, jnp.int32)])
    col = jnp.concatenate([ei[1], jnp.full((pad,), _N, jnp.int32)])
    row2d = row.reshape(_NCHUNK, _CHUNK)
    col2d = col.reshape(_NCHUNK, _CHUNK)
    x_pad = jnp.pad(x, ((0, _NP - _N), (0, 0)))
    eps_pad = jnp.pad(eps, ((0, _NP - _N), (0, 0)))
    b_mu8 = jnp.broadcast_to(b_mu[None, :], (8, _D))
    b_ls8 = jnp.broadcast_to(b_ls[None, :], (8, _D))

    deg1d = _deg(col2d)
    deg2d = deg1d.reshape(_NP // _D, _D)
    outs = _mm(x_pad, deg2d, W_mu, W_ls)
    xqs, dis_wide = outs[:2 * _NQ], outs[2 * _NQ]
    accs = _agg(xqs, row2d, col2d)
    z = _fin(dis_wide, accs, xqs, eps_pad, b_mu8, b_ls8)
    return z[:_N]
